# baseline (speedup 1.0000x reference)
import dataclasses
import functools

import jax
import jax.numpy as jnp
from jax import lax
from jax.experimental import pallas as pl
from jax.experimental.pallas import tpu as pltpu
from jax.experimental.pallas import tpu_sc as plsc

V = 100000
D = 32
B = 4096
FS = 16
FA = 6
FD = 4
L = 50
FC = FS + FD

NC = 2
NS = 16
NW = NC * NS

PB = B // NW
PAIRS = PB * FD
DF_ROWS = PB * L

CH = 128
DCH = 640
ACC_STRIDE = 520

_mesh = plsc.VectorSubcoreMesh(core_axis_name="c", subcore_axis_name="s")

_cparams = pltpu.CompilerParams()
if "needs_layout_passes" in pltpu.CompilerParams.__dataclass_fields__:
    _cparams = dataclasses.replace(_cparams, needs_layout_passes=False)
if "use_tc_tiling_on_sc" in pltpu.CompilerParams.__dataclass_fields__:
    _cparams = dataclasses.replace(_cparams, use_tc_tiling_on_sc=False)

_ci = functools.partial(lax.iota, jnp.int32)


def _lookup_body(nf, tab, ids, out, idx_v, dst_v, rows_v, ids_v, sem):
    c = lax.axis_index("c")
    s = lax.axis_index("s")
    b0 = (s * NC + c) * PB
    ci = _ci(16)

    pltpu.sync_copy(ids.at[pl.ds(b0, PB)], ids_v)

    @pl.loop(0, nf * 8)
    def _(m):
        pv = m * 16 + ci
        f = pv >> 7
        bl = pv & (PB - 1)
        plsc.store_scatter(idx_v, [pv], plsc.load_gather(ids_v, [bl, f]))
        plsc.store_scatter(dst_v, [jnp.full((16,), m // 8, jnp.int32),
                                   (m % 8) * 16 + ci],
                           (b0 + bl) * nf + f)

    cps = [pltpu.async_copy(tab.at[f].at[idx_v.at[pl.ds(f * CH, CH)]],
                            rows_v.at[pl.ds(f * CH, CH)], sem)
           for f in range(nf)]
    for cp in cps:
        cp.wait()
    for f in range(nf):
        pltpu.sync_copy(rows_v.at[pl.ds(f * CH, CH)], out.at[dst_v.at[f]])


def _dyn_body(dtab, dids, dlens, zeros, out,
              idx_v, seg_v, dst_v, rows_v, dids_v, lens_v, inv_v,
              shared, sem):
    c = lax.axis_index("c")
    s = lax.axis_index("s")
    wid = s * NC + c
    b0 = wid * PB
    ci = _ci(16)

    pltpu.sync_copy(dids.at[pl.ds(b0, PB)], dids_v)
    pltpu.sync_copy(dlens.at[pl.ds(b0, PB)], lens_v)
    pltpu.sync_copy(zeros, shared.at[pl.ds(s * ACC_STRIDE, ACC_STRIDE)])
    accbase = s * ACC_STRIDE

    for f in range(FD):
        for t in range(DF_ROWS // DCH):
            @pl.loop(0, DCH // 16)
            def _(m):
                qf = t * DCH + m * 16 + ci
                bl = qf // L
                ln = qf - bl * L
                pairl = bl * FD + f
                fv = jnp.full((16,), f, jnp.int32)
                idxv = plsc.load_gather(dids_v, [bl, fv, ln])
                plsc.store_scatter(idx_v, [m * 16 + ci], idxv)
                lenv = plsc.load_gather(lens_v, [bl, fv])
                segv = jnp.where(ln < lenv, pairl, PAIRS) + accbase
                plsc.store_scatter(seg_v, [jnp.full((16,), m // 8, jnp.int32),
                                           (m % 8) * 16 + ci], segv)

            cps = [pltpu.async_copy(dtab.at[f].at[idx_v.at[pl.ds(j * CH, CH)]],
                                    rows_v.at[pl.ds(j * CH, CH)], sem)
                   for j in range(DCH // CH)]
            for cp in cps:
                cp.wait()
            for j in range(DCH // CH):
                pltpu.sync_copy(rows_v.at[pl.ds(j * CH, CH)],
                                shared.at[seg_v.at[j]], add=True)

    for i in range(PAIRS // 16):
        pv = i * 16 + ci
        lf = plsc.load_gather(lens_v, [pv >> 2, pv & 3]).astype(jnp.float32)
        inv_v[pl.ds(i * 16, 16)] = 1.0 / jnp.maximum(lf, 1.0)

    pltpu.sync_copy(shared.at[pl.ds(accbase, PAIRS)],
                    rows_v.at[pl.ds(0, PAIRS)])

    @pl.loop(0, PAIRS)
    def _(p):
        rowi = jnp.full((16,), p, dtype=jnp.int32)
        invs = plsc.load_gather(inv_v, [rowi])
        for h in range(2):
            col = ci + h * 16
            v = plsc.load_gather(rows_v, [rowi, col])
            plsc.store_scatter(rows_v, [rowi, col], v * invs)

    pltpu.sync_copy(rows_v.at[pl.ds(0, PAIRS)],
                    out.at[pl.ds(b0 * FD, PAIRS)])


def _mk_lookup(nf):
    return pl.kernel(
        functools.partial(_lookup_body, nf),
        out_type=jax.ShapeDtypeStruct((B * nf, D), jnp.float32),
        mesh=_mesh,
        scratch_types=[
            pltpu.VMEM((nf * CH,), jnp.int32),
            pltpu.VMEM((nf, CH), jnp.int32),
            pltpu.VMEM((nf * CH, D), jnp.float32),
            pltpu.VMEM((PB, nf), jnp.int32),
            pltpu.SemaphoreType.DMA,
        ],
        compiler_params=_cparams,
    )


_static4_call = _mk_lookup(4)
_ad_call = _mk_lookup(FA)

_dyn_call = pl.kernel(
    _dyn_body,
    out_type=jax.ShapeDtypeStruct((B * FD, D), jnp.float32),
    mesh=_mesh,
    scratch_types=[
        pltpu.VMEM((DCH,), jnp.int32),
        pltpu.VMEM((8, CH), jnp.int32),
        pltpu.VMEM((4, CH), jnp.int32),
        pltpu.VMEM((DCH, D), jnp.float32),
        pltpu.VMEM((PB, FD, L), jnp.int32),
        pltpu.VMEM((PB, FD), jnp.int32),
        pltpu.VMEM((PAIRS,), jnp.float32),
        pltpu.VMEM_SHARED((NS * ACC_STRIDE, D), jnp.float32),
        pltpu.SemaphoreType.DMA,
    ],
    compiler_params=_cparams,
)


def kernel(static_ids, ad_ids, dynamic_ids, dynamic_lengths,
           static_tables, ad_tables, dynamic_tables):
    zeros = jnp.zeros((ACC_STRIDE, D), jnp.float32)
    aout = _ad_call(ad_tables, ad_ids)
    dout = _dyn_call(dynamic_tables, dynamic_ids, dynamic_lengths, zeros)
    souts = [
        _static4_call(
            lax.slice_in_dim(static_tables, g * 4, (g + 1) * 4, axis=0),
            lax.slice_in_dim(static_ids, g * 4, (g + 1) * 4, axis=1),
        ).reshape(B, 4, D)
        for g in range(4)
    ]
    out1 = jnp.concatenate(souts + [dout.reshape(B, FD, D)], axis=1)
    return (out1, aout.reshape(B, FA, D))


# speedup vs baseline: 1.4089x; 1.4089x over previous
import dataclasses
import functools

import jax
import jax.numpy as jnp
from jax import lax
from jax.experimental import pallas as pl
from jax.experimental.pallas import tpu as pltpu
from jax.experimental.pallas import tpu_sc as plsc

V = 100000
D = 32
B = 4096
FS = 16
FA = 6
FD = 4
L = 50
FC = FS + FD

NC = 2
NS = 16
NW = NC * NS

PB = B // NW
PAIRS = PB * FD
DF_ROWS = PB * L

CH = 128
DCH = 640
ACC_STRIDE = 520

_mesh = plsc.VectorSubcoreMesh(core_axis_name="c", subcore_axis_name="s")

_cparams = pltpu.CompilerParams()
if "needs_layout_passes" in pltpu.CompilerParams.__dataclass_fields__:
    _cparams = dataclasses.replace(_cparams, needs_layout_passes=False)
if "use_tc_tiling_on_sc" in pltpu.CompilerParams.__dataclass_fields__:
    _cparams = dataclasses.replace(_cparams, use_tc_tiling_on_sc=False)

_ci = functools.partial(lax.iota, jnp.int32)


def _lookup_body(nf, tab, ids, out, idx_v, dst_v, rows_v, ids_v, sem):
    c = lax.axis_index("c")
    s = lax.axis_index("s")
    b0 = (s * NC + c) * PB
    ci = _ci(16)

    pltpu.sync_copy(ids.at[pl.ds(b0, PB)], ids_v)

    @pl.loop(0, nf * 8)
    def _(m):
        pv = m * 16 + ci
        f = pv >> 7
        bl = pv & (PB - 1)
        plsc.store_scatter(idx_v, [pv], plsc.load_gather(ids_v, [bl, f]))
        plsc.store_scatter(dst_v, [jnp.full((16,), m // 8, jnp.int32),
                                   (m % 8) * 16 + ci],
                           (b0 + bl) * nf + f)

    cps = [pltpu.async_copy(tab.at[f].at[idx_v.at[pl.ds(f * CH, CH)]],
                            rows_v.at[pl.ds(f * CH, CH)], sem)
           for f in range(nf)]
    for cp in cps:
        cp.wait()
    for f in range(nf):
        pltpu.sync_copy(rows_v.at[pl.ds(f * CH, CH)], out.at[dst_v.at[f]])


def _dyn_body(dtab, dids, dlens, zeros, out,
              idx_v, seg_v, dst_v, rows_v, dids_v, lens_v, inv_v,
              shared, sem):
    c = lax.axis_index("c")
    s = lax.axis_index("s")
    wid = s * NC + c
    b0 = wid * PB
    ci = _ci(16)

    pltpu.sync_copy(dids.at[pl.ds(b0, PB)], dids_v)
    pltpu.sync_copy(dlens.at[pl.ds(b0, PB)], lens_v)
    pltpu.sync_copy(zeros, shared.at[pl.ds(s * ACC_STRIDE, ACC_STRIDE)])
    accbase = s * ACC_STRIDE

    for f in range(FD):
        for t in range(DF_ROWS // DCH):
            @pl.loop(0, DCH // 16)
            def _(m):
                qf = t * DCH + m * 16 + ci
                bl = qf // L
                ln = qf - bl * L
                pairl = bl * FD + f
                fv = jnp.full((16,), f, jnp.int32)
                idxv = plsc.load_gather(dids_v, [bl, fv, ln])
                plsc.store_scatter(idx_v, [m * 16 + ci], idxv)
                lenv = plsc.load_gather(lens_v, [bl, fv])
                segv = jnp.where(ln < lenv, pairl, PAIRS) + accbase
                plsc.store_scatter(seg_v, [jnp.full((16,), m // 8, jnp.int32),
                                           (m % 8) * 16 + ci], segv)

            cps = [pltpu.async_copy(dtab.at[f].at[idx_v.at[pl.ds(j * CH, CH)]],
                                    rows_v.at[pl.ds(j * CH, CH)], sem)
                   for j in range(DCH // CH)]
            for cp in cps:
                cp.wait()
            for j in range(DCH // CH):
                pltpu.sync_copy(rows_v.at[pl.ds(j * CH, CH)],
                                shared.at[seg_v.at[j]], add=True)

    for i in range(PAIRS // 16):
        pv = i * 16 + ci
        lf = plsc.load_gather(lens_v, [pv >> 2, pv & 3]).astype(jnp.float32)
        inv_v[pl.ds(i * 16, 16)] = 1.0 / jnp.maximum(lf, 1.0)

    pltpu.sync_copy(shared.at[pl.ds(accbase, PAIRS)],
                    rows_v.at[pl.ds(0, PAIRS)])

    @pl.loop(0, PAIRS)
    def _(p):
        rowi = jnp.full((16,), p, dtype=jnp.int32)
        invs = plsc.load_gather(inv_v, [rowi])
        for h in range(2):
            col = ci + h * 16
            v = plsc.load_gather(rows_v, [rowi, col])
            plsc.store_scatter(rows_v, [rowi, col], v * invs)

    pltpu.sync_copy(rows_v.at[pl.ds(0, PAIRS)],
                    out.at[pl.ds(b0 * FD, PAIRS)])


def _mk_lookup(nf):
    return pl.kernel(
        functools.partial(_lookup_body, nf),
        out_type=jax.ShapeDtypeStruct((B * nf, D), jnp.float32),
        mesh=_mesh,
        scratch_types=[
            pltpu.VMEM((nf * CH,), jnp.int32),
            pltpu.VMEM((nf, CH), jnp.int32),
            pltpu.VMEM((nf * CH, D), jnp.float32),
            pltpu.VMEM((PB, nf), jnp.int32),
            pltpu.SemaphoreType.DMA,
        ],
        compiler_params=_cparams,
    )


_static_call = _mk_lookup(FS)
_ad_call = _mk_lookup(FA)

_dyn_call = pl.kernel(
    _dyn_body,
    out_type=jax.ShapeDtypeStruct((B * FD, D), jnp.float32),
    mesh=_mesh,
    scratch_types=[
        pltpu.VMEM((DCH,), jnp.int32),
        pltpu.VMEM((8, CH), jnp.int32),
        pltpu.VMEM((4, CH), jnp.int32),
        pltpu.VMEM((DCH, D), jnp.float32),
        pltpu.VMEM((PB, FD, L), jnp.int32),
        pltpu.VMEM((PB, FD), jnp.int32),
        pltpu.VMEM((PAIRS,), jnp.float32),
        pltpu.VMEM_SHARED((NS * ACC_STRIDE, D), jnp.float32),
        pltpu.SemaphoreType.DMA,
    ],
    compiler_params=_cparams,
)


def kernel(static_ids, ad_ids, dynamic_ids, dynamic_lengths,
           static_tables, ad_tables, dynamic_tables):
    zeros = jnp.zeros((ACC_STRIDE, D), jnp.float32)
    aout = _ad_call(ad_tables, ad_ids)
    dout = _dyn_call(dynamic_tables, dynamic_ids, dynamic_lengths, zeros)
    sout = _static_call(static_tables, static_ids)
    out1 = jnp.concatenate(
        [sout.reshape(B, FS, D), dout.reshape(B, FD, D)], axis=1)
    return (out1, aout.reshape(B, FA, D))


# speedup vs baseline: 1.4090x; 1.0000x over previous
import dataclasses
import functools

import jax
import jax.numpy as jnp
from jax import lax
from jax.experimental import pallas as pl
from jax.experimental.pallas import tpu as pltpu
from jax.experimental.pallas import tpu_sc as plsc

V = 100000
D = 32
B = 4096
FS = 16
FA = 6
FD = 4
L = 50
FC = FS + FD

NC = 2
NS = 16
NW = NC * NS

PB = B // NW
PAIRS = PB * FD
DF_ROWS = PB * L

CH = 128
DCH = 640
ACC_STRIDE = 520

_mesh = plsc.VectorSubcoreMesh(core_axis_name="c", subcore_axis_name="s")

_cparams = pltpu.CompilerParams()
if "needs_layout_passes" in pltpu.CompilerParams.__dataclass_fields__:
    _cparams = dataclasses.replace(_cparams, needs_layout_passes=False)
if "use_tc_tiling_on_sc" in pltpu.CompilerParams.__dataclass_fields__:
    _cparams = dataclasses.replace(_cparams, use_tc_tiling_on_sc=False)

_ci = functools.partial(lax.iota, jnp.int32)


def _lookup_body(nf, tab, ids, out, idx_v, dst_v, rows_v, ids_v, sem):
    c = lax.axis_index("c")
    s = lax.axis_index("s")
    b0 = (s * NC + c) * PB
    ci = _ci(16)

    pltpu.sync_copy(ids.at[pl.ds(b0, PB)], ids_v)

    @pl.loop(0, nf * 8)
    def _(m):
        pv = m * 16 + ci
        f = pv >> 7
        bl = pv & (PB - 1)
        plsc.store_scatter(idx_v, [pv],
                           plsc.load_gather(ids_v, [bl, f]) + f * V)
        plsc.store_scatter(dst_v, [jnp.full((16,), m // 8, jnp.int32),
                                   (m % 8) * 16 + ci],
                           (b0 + bl) * nf + f)

    cps = [pltpu.async_copy(tab.at[idx_v.at[pl.ds(f * CH, CH)]],
                            rows_v.at[pl.ds(f * CH, CH)], sem)
           for f in range(nf)]
    for cp in cps:
        cp.wait()
    for f in range(nf):
        pltpu.sync_copy(rows_v.at[pl.ds(f * CH, CH)], out.at[dst_v.at[f]])


def _dyn_body(dtab, dids, dlens, zeros, out,
              idx_v, seg_v, dst_v, rows_v, dids_v, lens_v, inv_v,
              shared, sem):
    c = lax.axis_index("c")
    s = lax.axis_index("s")
    wid = s * NC + c
    b0 = wid * PB
    ci = _ci(16)

    pltpu.sync_copy(dids.at[pl.ds(b0, PB)], dids_v)
    pltpu.sync_copy(dlens.at[pl.ds(b0, PB)], lens_v)
    pltpu.sync_copy(zeros, shared.at[pl.ds(s * ACC_STRIDE, ACC_STRIDE)])
    accbase = s * ACC_STRIDE

    for f in range(FD):
        for t in range(DF_ROWS // DCH):
            @pl.loop(0, DCH // 16)
            def _(m):
                qf = t * DCH + m * 16 + ci
                bl = qf // L
                ln = qf - bl * L
                pairl = bl * FD + f
                fv = jnp.full((16,), f, jnp.int32)
                idxv = plsc.load_gather(dids_v, [bl, fv, ln]) + f * V
                plsc.store_scatter(idx_v, [m * 16 + ci], idxv)
                lenv = plsc.load_gather(lens_v, [bl, fv])
                segv = jnp.where(ln < lenv, pairl, PAIRS) + accbase
                plsc.store_scatter(seg_v, [jnp.full((16,), m // 8, jnp.int32),
                                           (m % 8) * 16 + ci], segv)

            cps = [pltpu.async_copy(dtab.at[idx_v.at[pl.ds(j * CH, CH)]],
                                    rows_v.at[pl.ds(j * CH, CH)], sem)
                   for j in range(DCH // CH)]
            for cp in cps:
                cp.wait()
            for j in range(DCH // CH):
                pltpu.sync_copy(rows_v.at[pl.ds(j * CH, CH)],
                                shared.at[seg_v.at[j]], add=True)

    for i in range(PAIRS // 16):
        pv = i * 16 + ci
        lf = plsc.load_gather(lens_v, [pv >> 2, pv & 3]).astype(jnp.float32)
        inv_v[pl.ds(i * 16, 16)] = 1.0 / jnp.maximum(lf, 1.0)

    pltpu.sync_copy(shared.at[pl.ds(accbase, PAIRS)],
                    rows_v.at[pl.ds(0, PAIRS)])

    @pl.loop(0, PAIRS)
    def _(p):
        rowi = jnp.full((16,), p, dtype=jnp.int32)
        invs = plsc.load_gather(inv_v, [rowi])
        for h in range(2):
            col = ci + h * 16
            v = plsc.load_gather(rows_v, [rowi, col])
            plsc.store_scatter(rows_v, [rowi, col], v * invs)

    pltpu.sync_copy(rows_v.at[pl.ds(0, PAIRS)],
                    out.at[pl.ds(b0 * FD, PAIRS)])


def _mk_lookup(nf):
    return pl.kernel(
        functools.partial(_lookup_body, nf),
        out_type=jax.ShapeDtypeStruct((B * nf, D), jnp.float32),
        mesh=_mesh,
        scratch_types=[
            pltpu.VMEM((nf * CH,), jnp.int32),
            pltpu.VMEM((nf, CH), jnp.int32),
            pltpu.VMEM((nf * CH, D), jnp.float32),
            pltpu.VMEM((PB, nf), jnp.int32),
            pltpu.SemaphoreType.DMA,
        ],
        compiler_params=_cparams,
    )


_static_call = _mk_lookup(FS)
_ad_call = _mk_lookup(FA)

_dyn_call = pl.kernel(
    _dyn_body,
    out_type=jax.ShapeDtypeStruct((B * FD, D), jnp.float32),
    mesh=_mesh,
    scratch_types=[
        pltpu.VMEM((DCH,), jnp.int32),
        pltpu.VMEM((8, CH), jnp.int32),
        pltpu.VMEM((4, CH), jnp.int32),
        pltpu.VMEM((DCH, D), jnp.float32),
        pltpu.VMEM((PB, FD, L), jnp.int32),
        pltpu.VMEM((PB, FD), jnp.int32),
        pltpu.VMEM((PAIRS,), jnp.float32),
        pltpu.VMEM_SHARED((NS * ACC_STRIDE, D), jnp.float32),
        pltpu.SemaphoreType.DMA,
    ],
    compiler_params=_cparams,
)


def kernel(static_ids, ad_ids, dynamic_ids, dynamic_lengths,
           static_tables, ad_tables, dynamic_tables):
    zeros = jnp.zeros((ACC_STRIDE, D), jnp.float32)
    aout = _ad_call(ad_tables.reshape(FA * V, D), ad_ids)
    dout = _dyn_call(dynamic_tables.reshape(FD * V, D), dynamic_ids,
                     dynamic_lengths, zeros)
    sout = _static_call(static_tables.reshape(FS * V, D), static_ids)
    out1 = jnp.concatenate(
        [sout.reshape(B, FS, D), dout.reshape(B, FD, D)], axis=1)
    return (out1, aout.reshape(B, FA, D))
